# baseline (speedup 1.0000x reference)
import functools

import jax
import jax.numpy as jnp
from jax import lax
from jax.experimental import pallas as pl
from jax.experimental.pallas import tpu as pltpu
from jax.experimental.pallas import tpu_sc as plsc

N = 50000
NN = 2 * N
CS = 12544
NQ = 8
NP = CS * NQ
E = 800000
E2 = 2 * E
HID = 64
W128 = 128
EMBP = 128
NC, NS = 2, 16
NW = NC * NS
EW = E2 // NW
B1 = 2000
STG = 2176
NBCAP = EW // 128 + 1
CAP = NBCAP * 128
ACC = CS + 128
ZROWS = ACC // NS
WROWS = CS // NS

_mesh = plsc.VectorSubcoreMesh(core_axis_name="c", subcore_axis_name="s")
_f32 = jnp.float32
_i32 = jnp.int32


def _iota16():
    return lax.iota(_i32, 16)


@functools.partial(
    pl.kernel,
    out_type=(
        jax.ShapeDtypeStruct((NQ, NW, CAP), _i32),
        jax.ShapeDtypeStruct((NQ, NW, CAP), _i32),
        jax.ShapeDtypeStruct((NW, 16), _i32),
    ),
    mesh=_mesh,
    scratch_types=[
        pltpu.VMEM((B1,), _i32),
        pltpu.VMEM((B1,), _i32),
        pltpu.VMEM((NQ * STG,), _i32),
        pltpu.VMEM((NQ * STG,), _i32),
        pltpu.VMEM((16,), _i32),
        pltpu.VMEM((32,), _i32),
    ],
)
def _sc_bucketize(src_hbm, dst_hbm, srcl, dstl, counts, sbuf, dbuf, stage_s, stage_d, cntbuf, pbuf):
    c = lax.axis_index("c")
    s = lax.axis_index("s")
    w = s * NC + c
    base = w * EW
    pbuf[pl.ds(0, 16)] = jnp.zeros((16,), _i32)

    def block_body(b, carry):
        offs, nbs = carry
        pltpu.sync_copy(src_hbm.at[pl.ds(base + b * B1, B1)], sbuf)
        pltpu.sync_copy(dst_hbm.at[pl.ds(base + b * B1, B1)], dbuf)
        new_offs, new_nbs = [], []
        for q in range(NQ):
            lo = q * CS

            def vloop(i, off, lo=lo, q=q):
                d = dbuf[pl.ds(i * 16, 16)]
                sv = sbuf[pl.ds(i * 16, 16)]
                m = (d >= lo) & (d < lo + CS)
                p = m.astype(_i32)
                for k2 in (1, 2, 4, 8):
                    pbuf[pl.ds(16, 16)] = p
                    p = p + pbuf[pl.ds(16 - k2, 16)]
                idx = jnp.where(m, q * STG + off + (p - 1),
                                q * STG + STG - 16 + _iota16())
                plsc.store_scatter(stage_s, [idx], sv)
                plsc.store_scatter(stage_d, [idx], d - lo)
                return off + p[15]

            off_q = lax.fori_loop(0, B1 // 16, vloop, offs[q])
            nfl = off_q // 128

            def flush(k, _, q=q, nb0=nbs[q]):
                pltpu.sync_copy(stage_s.at[pl.ds(q * STG + k * 128, 128)],
                                srcl.at[q, w, pl.ds((nb0 + k) * 128, 128)])
                pltpu.sync_copy(stage_d.at[pl.ds(q * STG + k * 128, 128)],
                                dstl.at[q, w, pl.ds((nb0 + k) * 128, 128)])
                return 0

            lax.fori_loop(0, nfl, flush, 0)

            @pl.when(nfl > 0)
            def _(q=q, nfl=nfl):
                def mv(i, _):
                    stage_s[pl.ds(q * STG + i * 16, 16)] = stage_s[pl.ds(q * STG + nfl * 128 + i * 16, 16)]
                    stage_d[pl.ds(q * STG + i * 16, 16)] = stage_d[pl.ds(q * STG + nfl * 128 + i * 16, 16)]
                    return 0
                lax.fori_loop(0, 8, mv, 0)

            new_offs.append(off_q - nfl * 128)
            new_nbs.append(nbs[q] + nfl)
        return tuple(new_offs), tuple(new_nbs)

    zero = jnp.zeros((), _i32)
    offs, nbs = lax.fori_loop(0, EW // B1, block_body,
                              ((zero,) * NQ, (zero,) * NQ))

    cnts = jnp.zeros((16,), _i32)
    padsrc = jnp.zeros((16,), _i32)
    paddst = jnp.full((16,), CS, _i32)
    for q in range(NQ):
        off = offs[q]
        for k in range(8):
            stage_s[pl.ds(q * STG + off + k * 16, 16)] = padsrc
            stage_d[pl.ds(q * STG + off + k * 16, 16)] = paddst

        @pl.when(off > 0)
        def _(q=q, nb=nbs[q]):
            pltpu.sync_copy(stage_s.at[pl.ds(q * STG, 128)],
                            srcl.at[q, w, pl.ds(nb * 128, 128)])
            pltpu.sync_copy(stage_d.at[pl.ds(q * STG, 128)],
                            dstl.at[q, w, pl.ds(nb * 128, 128)])

        cnts = jnp.where(_iota16() == q, jnp.where(off > 0, nbs[q] + 1, nbs[q]), cnts)
    cntbuf[...] = cnts
    pltpu.sync_copy(cntbuf, counts.at[w])


@functools.partial(
    pl.kernel,
    out_type=jax.ShapeDtypeStruct((NP, W128), _f32),
    mesh=_mesh,
    scratch_types=[
        pltpu.VMEM_SHARED((ACC, W128), _f32),
        pltpu.VMEM((128,), _i32),
        pltpu.VMEM((1, 128), _i32),
        pltpu.VMEM((128, W128), _f32),
        pltpu.VMEM((16,), _i32),
        pltpu.SemaphoreType.DMA,
        pltpu.SemaphoreType.DMA,
    ],
)
def _sc_aggregate(h, srcl, dstl3, counts, zb, agg, accum, idxbuf, dstbuf, rows, cbuf, gsem, ssem):
    c = lax.axis_index("c")
    s = lax.axis_index("s")
    for qi in range(NQ // NC):
        q = c * (NQ // NC) + qi
        pltpu.sync_copy(zb, accum.at[pl.ds(s * ZROWS, ZROWS)])
        plsc.subcore_barrier()
        for wi in range(NW // NS):
            w = s + NS * wi
            pltpu.sync_copy(counts.at[w], cbuf)
            cv = cbuf[...]
            nb = jnp.zeros((), _i32)
            for lane in range(NQ):
                nb = jnp.where(q == lane, cv[lane], nb)

            def blk(j, _, q=q, w=w):
                pltpu.sync_copy(srcl.at[q, w, pl.ds(j * 128, 128)], idxbuf)
                pltpu.sync_copy(dstl3.at[q, w, j], dstbuf.at[0])
                pltpu.async_copy(h.at[idxbuf], rows, gsem).wait()
                pltpu.async_copy(rows, accum.at[dstbuf.at[0]], ssem, add=True).wait()
                return 0

            lax.fori_loop(0, nb, blk, 0)
        plsc.subcore_barrier()
        pltpu.sync_copy(accum.at[pl.ds(s * WROWS, WROWS)],
                        agg.at[pl.ds(q * CS + s * WROWS, WROWS)])
        plsc.subcore_barrier()


@functools.partial(
    pl.kernel,
    out_type=jax.ShapeDtypeStruct((NP, W128), _f32),
    mesh=_mesh,
    scratch_types=[
        pltpu.VMEM_SHARED((ACC, W128), _f32),
        pltpu.VMEM((1, 128), _i32),
        pltpu.VMEM((128, W128), _f32),
        pltpu.VMEM((16,), _i32),
        pltpu.SemaphoreType.DMA,
    ],
)
def _sc_degree(dstl3, counts, zb, ones_hbm, deg, accum, dstbuf, onesbuf, cbuf, ssem):
    c = lax.axis_index("c")
    s = lax.axis_index("s")
    pltpu.sync_copy(ones_hbm, onesbuf)
    for qi in range(NQ // NC):
        q = c * (NQ // NC) + qi
        pltpu.sync_copy(zb, accum.at[pl.ds(s * ZROWS, ZROWS)])
        plsc.subcore_barrier()
        for wi in range(NW // NS):
            w = s + NS * wi
            pltpu.sync_copy(counts.at[w], cbuf)
            cv = cbuf[...]
            nb = jnp.zeros((), _i32)
            for lane in range(NQ):
                nb = jnp.where(q == lane, cv[lane], nb)

            def blk(j, _, q=q, w=w):
                pltpu.sync_copy(dstl3.at[q, w, j], dstbuf.at[0])
                pltpu.async_copy(onesbuf, accum.at[dstbuf.at[0]], ssem, add=True).wait()
                return 0

            lax.fori_loop(0, nb, blk, 0)
        plsc.subcore_barrier()
        pltpu.sync_copy(accum.at[pl.ds(s * WROWS, WROWS)],
                        deg.at[pl.ds(q * CS + s * WROWS, WROWS)])
        plsc.subcore_barrier()


@functools.partial(
    pl.kernel,
    out_type=jax.ShapeDtypeStruct((NP, EMBP), _f32),
    mesh=_mesh,
    scratch_types=[
        pltpu.VMEM((112,), _i32),
        pltpu.VMEM((112, EMBP), _f32),
        pltpu.SemaphoreType.DMA,
    ],
)
def _sc_gather_emb(embp, eidx2, xg, ibuf, rows, gsem):
    c = lax.axis_index("c")
    s = lax.axis_index("s")
    w = s * NC + c

    def blk(j, _):
        r = w * 28 + j
        pltpu.sync_copy(eidx2.at[r], ibuf)
        pltpu.async_copy(embp.at[ibuf], rows, gsem).wait()
        pltpu.sync_copy(rows, xg.at[pl.ds(r * 112, 112)])
        return 0

    lax.fori_loop(0, 28, blk, 0)


@functools.partial(
    pl.kernel,
    out_type=jax.ShapeDtypeStruct((2048, W128), _f32),
    mesh=_mesh,
    scratch_types=[
        pltpu.VMEM((64,), _i32),
        pltpu.VMEM((64, W128), _f32),
        pltpu.SemaphoreType.DMA,
    ],
)
def _sc_gather_rows(h, pidx2, geg, ibuf, rows, gsem):
    c = lax.axis_index("c")
    s = lax.axis_index("s")
    w = s * NC + c
    pltpu.sync_copy(pidx2.at[w], ibuf)
    pltpu.async_copy(h.at[ibuf], rows, gsem).wait()
    pltpu.sync_copy(rows, geg.at[pl.ds(w * 64, 64)])


def _t0_body(xg_ref, pwt_ref, pb_ref, o_ref):
    x = jnp.dot(xg_ref[...], pwt_ref[...], preferred_element_type=_f32) + pb_ref[...]
    o_ref[...] = jnp.concatenate([x, jnp.zeros((x.shape[0], W128 - HID), _f32)], axis=1)


def _t1_body(agg_ref, h_ref, dg_ref, wt_ref, b_ref, wih_ref, bih_ref, whh_ref, bhh_ref, o_ref):
    h = h_ref[...][:, :HID]
    inc = jnp.dot(agg_ref[...], wt_ref[...], preferred_element_type=_f32)
    inc = inc + dg_ref[...][:, :HID] * b_ref[...]
    gi = jnp.dot(inc, wih_ref[...], preferred_element_type=_f32) + bih_ref[...]
    gh = jnp.dot(h, whh_ref[...], preferred_element_type=_f32) + bhh_ref[...]
    r = jax.nn.sigmoid(gi[:, :HID] + gh[:, :HID])
    z = jax.nn.sigmoid(gi[:, HID:2 * HID] + gh[:, HID:2 * HID])
    n = jnp.tanh(gi[:, 2 * HID:] + r * gh[:, 2 * HID:])
    hn = (1.0 - z) * n + z * h
    o_ref[...] = jnp.concatenate([hn, jnp.zeros((hn.shape[0], W128 - HID), _f32)], axis=1)


def _tf_body(ge_ref, w1_ref, b1_ref, w2_ref, b2_ref, lab_ref, logit_ref, loss_ref):
    ge = ge_ref[...]
    hc = jnp.maximum(jnp.dot(ge, w1_ref[...], preferred_element_type=_f32) + b1_ref[...], 0.0)
    lg = jnp.dot(hc, w2_ref[...], preferred_element_type=_f32) + b2_ref[...]
    pv = jax.nn.sigmoid(lg)
    logit_ref[...] = pv
    lab = lab_ref[...]
    p = jnp.clip(pv, 1e-7, 1.0 - 1e-7)
    terms = lab * jnp.log(p) + (1.0 - lab) * jnp.log(1.0 - p)
    loss_ref[0, 0] = -jnp.sum(terms) / 1024.0


_GRID = NP // 1024


def _row_spec(width):
    return pl.BlockSpec((1024, width), lambda i: (i, 0))


def _full_spec(shape):
    return pl.BlockSpec(shape, lambda i: tuple(0 for _ in shape))


_t0_call = pl.pallas_call(
    _t0_body,
    grid=(_GRID,),
    in_specs=[_row_spec(EMBP), _full_spec((EMBP, HID)), _full_spec((1, HID))],
    out_specs=_row_spec(W128),
    out_shape=jax.ShapeDtypeStruct((NP, W128), _f32),
)

_t1_call = pl.pallas_call(
    _t1_body,
    grid=(_GRID,),
    in_specs=[
        _row_spec(W128), _row_spec(W128), _row_spec(W128),
        _full_spec((W128, HID)), _full_spec((1, HID)),
        _full_spec((HID, 3 * HID)), _full_spec((1, 3 * HID)),
        _full_spec((HID, 3 * HID)), _full_spec((1, 3 * HID)),
    ],
    out_specs=_row_spec(W128),
    out_shape=jax.ShapeDtypeStruct((NP, W128), _f32),
)

_tf_call = pl.pallas_call(
    _tf_body,
    in_specs=[
        pl.BlockSpec(memory_space=pltpu.VMEM),
        pl.BlockSpec(memory_space=pltpu.VMEM),
        pl.BlockSpec(memory_space=pltpu.VMEM),
        pl.BlockSpec(memory_space=pltpu.VMEM),
        pl.BlockSpec(memory_space=pltpu.VMEM),
        pl.BlockSpec(memory_space=pltpu.VMEM),
    ],
    out_specs=(
        pl.BlockSpec(memory_space=pltpu.VMEM),
        pl.BlockSpec(memory_space=pltpu.SMEM),
    ),
    out_shape=(
        jax.ShapeDtypeStruct((1024, 1), _f32),
        jax.ShapeDtypeStruct((1, 1), _f32),
    ),
)


def kernel(emb_ind_0, emb_ind_1, adj_0, adj_1, prop_ind_0, prop_ind_1, labels,
           emb_table, proj_W, proj_b, msg_W, msg_b, gru_Wih, gru_Whh, gru_bih, gru_bhh,
           cla_W1, cla_b1, cla_W2, cla_b2):
    src = jnp.concatenate([adj_0[:, 0], adj_1[:, 0] + N])
    dst = jnp.concatenate([adj_0[:, 1], adj_1[:, 1] + N])
    emb_ind = jnp.concatenate([emb_ind_0, emb_ind_1])
    embp = jnp.pad(emb_table, ((0, 0), (0, EMBP - emb_table.shape[1])))
    pwt = jnp.pad(proj_W.T, ((0, EMBP - proj_W.shape[1]), (0, 0)))

    xg = jnp.take(embp, jnp.pad(emb_ind, (0, NP - NN)), axis=0)
    h = _t0_call(xg, pwt, proj_b.reshape(1, HID))
    deg1 = jnp.zeros((NP,), _f32).at[dst].add(1.0)
    deg = jnp.broadcast_to(deg1[:, None], (NP, W128))

    for l in range(2):
        wt = jnp.pad(msg_W[l].T, ((0, W128 - HID), (0, 0)))
        b = msg_b[l].reshape(1, HID)
        wih = gru_Wih[l].T
        bih = gru_bih[l].reshape(1, 3 * HID)
        whh = gru_Whh[l].T
        bhh = gru_bhh[l].reshape(1, 3 * HID)
        for _ in range(3):
            agg = jnp.zeros((NP, W128), _f32).at[dst].add(h[src])
            h = _t1_call(agg, h, deg, wt, b, wih, bih, whh, bhh)

    ge = jnp.concatenate([h[prop_ind_0, :HID], h[prop_ind_1 + N, :HID]], axis=1)

    logits, loss = _tf_call(
        ge, cla_W1.T, cla_b1.reshape(1, HID), cla_W2.T, cla_b2.reshape(1, 1),
        labels.astype(_f32).reshape(1024, 1),
    )
    return logits, loss.reshape(())


# speedup vs baseline: 4.5905x; 4.5905x over previous
import functools

import jax
import jax.numpy as jnp
from jax import lax
from jax.experimental import pallas as pl
from jax.experimental.pallas import tpu as pltpu
from jax.experimental.pallas import tpu_sc as plsc

N = 50000
NN = 2 * N
CS = 12544
NQ = 8
NP = CS * NQ
E = 800000
E2 = 2 * E
HID = 64
W128 = 128
EMBP = 128
NC, NS = 2, 16
NW = NC * NS
EW = E2 // NW
B1 = 2000
STG = 2176
CAP = 50176
NBCAP = CAP // 128
ACC = CS + 128
ZROWS = ACC // NS
WROWS = CS // NS

_mesh = plsc.VectorSubcoreMesh(core_axis_name="c", subcore_axis_name="s")
_f32 = jnp.float32
_i32 = jnp.int32


def _iota16():
    return lax.iota(_i32, 16)


@functools.partial(
    pl.kernel,
    out_type=jax.ShapeDtypeStruct((NP, W128), _f32),
    mesh=_mesh,
    scratch_types=[
        pltpu.VMEM_SHARED((ACC, W128), _f32),
        pltpu.VMEM((128,), _i32),
        pltpu.VMEM((128,), _i32),
        pltpu.VMEM((128,), _i32),
        pltpu.VMEM((128, W128), _f32),
        pltpu.VMEM((16,), _i32),
        pltpu.SemaphoreType.DMA,
        pltpu.SemaphoreType.DMA,
    ],
)
def _sc_aggregate(h, ssrc, sdst, qtab, zb, agg, accum, idxbuf, dbuf128, dstbuf, rows, cbuf, gsem, ssem):
    c = lax.axis_index("c")
    s = lax.axis_index("s")
    pltpu.sync_copy(qtab.at[0], cbuf)
    sv = cbuf[...]
    pltpu.sync_copy(qtab.at[1], cbuf)
    nv = cbuf[...]

    def chunk(qi, _):
        q = c * (NQ // NC) + qi
        sb = jnp.zeros((), _i32)
        nb = jnp.zeros((), _i32)
        for lane in range(NQ):
            sb = jnp.where(q == lane, sv[lane], sb)
            nb = jnp.where(q == lane, nv[lane], nb)
        pltpu.sync_copy(zb, accum.at[pl.ds(s * ZROWS, ZROWS)])
        plsc.subcore_barrier()
        lo = q * CS

        def blk(k, _):
            j = sb + s + 16 * k
            pltpu.sync_copy(ssrc.at[pl.ds(j * 128, 128)], idxbuf)
            pltpu.sync_copy(sdst.at[pl.ds(j * 128, 128)], dbuf128)
            for k8 in range(8):
                d = dbuf128[pl.ds(k8 * 16, 16)]
                dl = d - lo
                ok = (dl >= 0) & (dl < CS)
                dstbuf[pl.ds(k8 * 16, 16)] = jnp.where(ok, dl, CS)
            pltpu.async_copy(h.at[idxbuf], rows, gsem).wait()
            pltpu.async_copy(rows, accum.at[dstbuf], ssem, add=True).wait()
            return 0

        nk = (nb - s + 15) // 16
        lax.fori_loop(0, nk, blk, 0)
        plsc.subcore_barrier()
        pltpu.sync_copy(accum.at[pl.ds(s * WROWS, WROWS)],
                        agg.at[pl.ds(q * CS + s * WROWS, WROWS)])
        plsc.subcore_barrier()
        return 0

    lax.fori_loop(0, NQ // NC, chunk, 0)


@functools.partial(
    pl.kernel,
    out_type=jax.ShapeDtypeStruct((NP, W128), _f32),
    mesh=_mesh,
    scratch_types=[
        pltpu.VMEM_SHARED((ACC, W128), _f32),
        pltpu.VMEM((128,), _i32),
        pltpu.VMEM((128,), _i32),
        pltpu.VMEM((128, W128), _f32),
        pltpu.VMEM((16,), _i32),
        pltpu.SemaphoreType.DMA,
    ],
)
def _sc_degree(sdst, qtab, zb, ones_hbm, deg, accum, dbuf128, dstbuf, onesbuf, cbuf, ssem):
    c = lax.axis_index("c")
    s = lax.axis_index("s")
    pltpu.sync_copy(ones_hbm, onesbuf)
    pltpu.sync_copy(qtab.at[0], cbuf)
    sv = cbuf[...]
    pltpu.sync_copy(qtab.at[1], cbuf)
    nv = cbuf[...]

    def chunk(qi, _):
        q = c * (NQ // NC) + qi
        sb = jnp.zeros((), _i32)
        nb = jnp.zeros((), _i32)
        for lane in range(NQ):
            sb = jnp.where(q == lane, sv[lane], sb)
            nb = jnp.where(q == lane, nv[lane], nb)
        pltpu.sync_copy(zb, accum.at[pl.ds(s * ZROWS, ZROWS)])
        plsc.subcore_barrier()
        lo = q * CS

        def blk(k, _):
            j = sb + s + 16 * k
            pltpu.sync_copy(sdst.at[pl.ds(j * 128, 128)], dbuf128)
            for k8 in range(8):
                d = dbuf128[pl.ds(k8 * 16, 16)]
                dl = d - lo
                ok = (dl >= 0) & (dl < CS)
                dstbuf[pl.ds(k8 * 16, 16)] = jnp.where(ok, dl, CS)
            pltpu.async_copy(onesbuf, accum.at[dstbuf], ssem, add=True).wait()
            return 0

        nk = (nb - s + 15) // 16
        lax.fori_loop(0, nk, blk, 0)
        plsc.subcore_barrier()
        pltpu.sync_copy(accum.at[pl.ds(s * WROWS, WROWS)],
                        deg.at[pl.ds(q * CS + s * WROWS, WROWS)])
        plsc.subcore_barrier()
        return 0

    lax.fori_loop(0, NQ // NC, chunk, 0)


@functools.partial(
    pl.kernel,
    out_type=jax.ShapeDtypeStruct((NP, EMBP), _f32),
    mesh=_mesh,
    scratch_types=[
        pltpu.VMEM((112,), _i32),
        pltpu.VMEM((112, EMBP), _f32),
        pltpu.SemaphoreType.DMA,
    ],
)
def _sc_gather_emb(embp, eidx2, xg, ibuf, rows, gsem):
    c = lax.axis_index("c")
    s = lax.axis_index("s")
    w = s * NC + c

    def blk(j, _):
        r = w * 28 + j
        pltpu.sync_copy(eidx2.at[r], ibuf)
        pltpu.async_copy(embp.at[ibuf], rows, gsem).wait()
        pltpu.sync_copy(rows, xg.at[pl.ds(r * 112, 112)])
        return 0

    lax.fori_loop(0, 28, blk, 0)


@functools.partial(
    pl.kernel,
    out_type=jax.ShapeDtypeStruct((2048, W128), _f32),
    mesh=_mesh,
    scratch_types=[
        pltpu.VMEM((64,), _i32),
        pltpu.VMEM((64, W128), _f32),
        pltpu.SemaphoreType.DMA,
    ],
)
def _sc_gather_rows(h, pidx2, geg, ibuf, rows, gsem):
    c = lax.axis_index("c")
    s = lax.axis_index("s")
    w = s * NC + c
    pltpu.sync_copy(pidx2.at[w], ibuf)
    pltpu.async_copy(h.at[ibuf], rows, gsem).wait()
    pltpu.sync_copy(rows, geg.at[pl.ds(w * 64, 64)])


def _t0_body(xg_ref, pwt_ref, pb_ref, o_ref):
    x = jnp.dot(xg_ref[...], pwt_ref[...], preferred_element_type=_f32) + pb_ref[...]
    o_ref[...] = jnp.concatenate([x, jnp.zeros((x.shape[0], W128 - HID), _f32)], axis=1)


def _t1_body(agg_ref, h_ref, dg_ref, wt_ref, b_ref, wih_ref, bih_ref, whh_ref, bhh_ref, o_ref):
    h = h_ref[...][:, :HID]
    inc = jnp.dot(agg_ref[...], wt_ref[...], preferred_element_type=_f32)
    inc = inc + dg_ref[...][:, :HID] * b_ref[...]
    gi = jnp.dot(inc, wih_ref[...], preferred_element_type=_f32) + bih_ref[...]
    gh = jnp.dot(h, whh_ref[...], preferred_element_type=_f32) + bhh_ref[...]
    r = jax.nn.sigmoid(gi[:, :HID] + gh[:, :HID])
    z = jax.nn.sigmoid(gi[:, HID:2 * HID] + gh[:, HID:2 * HID])
    n = jnp.tanh(gi[:, 2 * HID:] + r * gh[:, 2 * HID:])
    hn = (1.0 - z) * n + z * h
    o_ref[...] = jnp.concatenate([hn, jnp.zeros((hn.shape[0], W128 - HID), _f32)], axis=1)


def _tf_body(ge_ref, w1_ref, b1_ref, w2_ref, b2_ref, lab_ref, logit_ref, loss_ref):
    ge = ge_ref[...]
    hc = jnp.maximum(jnp.dot(ge, w1_ref[...], preferred_element_type=_f32) + b1_ref[...], 0.0)
    lg = jnp.dot(hc, w2_ref[...], preferred_element_type=_f32) + b2_ref[...]
    pv = jax.nn.sigmoid(lg)
    logit_ref[...] = pv
    lab = lab_ref[...]
    p = jnp.clip(pv, 1e-7, 1.0 - 1e-7)
    terms = lab * jnp.log(p) + (1.0 - lab) * jnp.log(1.0 - p)
    loss_ref[0, 0] = -jnp.sum(terms) / 1024.0


_GRID = NP // 1024


def _row_spec(width):
    return pl.BlockSpec((1024, width), lambda i: (i, 0))


def _full_spec(shape):
    return pl.BlockSpec(shape, lambda i: tuple(0 for _ in shape))


_t0_call = pl.pallas_call(
    _t0_body,
    grid=(_GRID,),
    in_specs=[_row_spec(EMBP), _full_spec((EMBP, HID)), _full_spec((1, HID))],
    out_specs=_row_spec(W128),
    out_shape=jax.ShapeDtypeStruct((NP, W128), _f32),
)

_t1_call = pl.pallas_call(
    _t1_body,
    grid=(_GRID,),
    in_specs=[
        _row_spec(W128), _row_spec(W128), _row_spec(W128),
        _full_spec((W128, HID)), _full_spec((1, HID)),
        _full_spec((HID, 3 * HID)), _full_spec((1, 3 * HID)),
        _full_spec((HID, 3 * HID)), _full_spec((1, 3 * HID)),
    ],
    out_specs=_row_spec(W128),
    out_shape=jax.ShapeDtypeStruct((NP, W128), _f32),
)

_tf_call = pl.pallas_call(
    _tf_body,
    in_specs=[
        pl.BlockSpec(memory_space=pltpu.VMEM),
        pl.BlockSpec(memory_space=pltpu.VMEM),
        pl.BlockSpec(memory_space=pltpu.VMEM),
        pl.BlockSpec(memory_space=pltpu.VMEM),
        pl.BlockSpec(memory_space=pltpu.VMEM),
        pl.BlockSpec(memory_space=pltpu.VMEM),
    ],
    out_specs=(
        pl.BlockSpec(memory_space=pltpu.VMEM),
        pl.BlockSpec(memory_space=pltpu.SMEM),
    ),
    out_shape=(
        jax.ShapeDtypeStruct((1024, 1), _f32),
        jax.ShapeDtypeStruct((1, 1), _f32),
    ),
)


def kernel(emb_ind_0, emb_ind_1, adj_0, adj_1, prop_ind_0, prop_ind_1, labels,
           emb_table, proj_W, proj_b, msg_W, msg_b, gru_Wih, gru_Whh, gru_bih, gru_bhh,
           cla_W1, cla_b1, cla_W2, cla_b2):
    src = jnp.concatenate([adj_0[:, 0], adj_1[:, 0] + N])
    dst = jnp.concatenate([adj_0[:, 1], adj_1[:, 1] + N])
    emb_ind = jnp.concatenate([emb_ind_0, emb_ind_1])
    embp = jnp.pad(emb_table, ((0, 0), (0, EMBP - emb_table.shape[1])))
    pwt = jnp.pad(proj_W.T, ((0, EMBP - proj_W.shape[1]), (0, 0)))

    eidx2 = jnp.pad(emb_ind, (0, NP - NN)).reshape(NP // 112, 112)
    zb = jnp.zeros((ZROWS, W128), _f32)
    ones = jnp.ones((128, W128), _f32)

    order = jnp.argsort(dst)
    ssrc = jnp.take(src, order)
    sdst = jnp.take(dst, order)
    b = jnp.searchsorted(sdst, jnp.arange(NQ + 1, dtype=_i32) * CS)
    startblk = (b[:NQ] // 128).astype(_i32)
    nblk = ((b[1:] + 127) // 128).astype(_i32) - startblk
    qtab = jnp.stack([jnp.pad(startblk, (0, 8)), jnp.pad(nblk, (0, 8))])

    deg = _sc_degree(sdst, qtab, zb, ones)
    xg = _sc_gather_emb(embp, eidx2)
    h = _t0_call(xg, pwt, proj_b.reshape(1, HID))

    for l in range(2):
        wt = jnp.pad(msg_W[l].T, ((0, W128 - HID), (0, 0)))
        b = msg_b[l].reshape(1, HID)
        wih = gru_Wih[l].T
        bih = gru_bih[l].reshape(1, 3 * HID)
        whh = gru_Whh[l].T
        bhh = gru_bhh[l].reshape(1, 3 * HID)
        for _ in range(3):
            agg = _sc_aggregate(h, ssrc, sdst, qtab, zb)
            h = _t1_call(agg, h, deg, wt, b, wih, bih, whh, bhh)

    pidx2 = jnp.concatenate([prop_ind_0, prop_ind_1 + N]).reshape(NW, 64)
    geg = _sc_gather_rows(h, pidx2)
    ge = jnp.concatenate([geg[:1024, :HID], geg[1024:, :HID]], axis=1)

    logits, loss = _tf_call(
        ge, cla_W1.T, cla_b1.reshape(1, HID), cla_W2.T, cla_b2.reshape(1, 1),
        labels.astype(_f32).reshape(1024, 1),
    )
    return logits, loss.reshape(())


# speedup vs baseline: 5.9597x; 1.2983x over previous
import functools

import jax
import jax.numpy as jnp
from jax import lax
from jax.experimental import pallas as pl
from jax.experimental.pallas import tpu as pltpu
from jax.experimental.pallas import tpu_sc as plsc

N = 50000
NN = 2 * N
CS = 6272
NQ = 16
NP = CS * NQ
E = 800000
E2 = 2 * E
HID = 64
W128 = 128
EMBP = 128
NC, NS = 2, 16
NW = NC * NS
EW = E2 // NW
B1 = 2000
STG = 2176
CAP = 50176
NBCAP = CAP // 128
ACC = CS + 128
ZROWS = ACC // NS
WROWS = CS // NS

_mesh = plsc.VectorSubcoreMesh(core_axis_name="c", subcore_axis_name="s")
_f32 = jnp.float32
_i32 = jnp.int32


def _iota16():
    return lax.iota(_i32, 16)


@functools.partial(
    pl.kernel,
    out_type=jax.ShapeDtypeStruct((NP, W128), _f32),
    mesh=_mesh,
    scratch_types=[
        pltpu.VMEM_SHARED((ACC, W128), _f32),
        pltpu.VMEM((4, 128), _i32),
        pltpu.VMEM((128,), _i32),
        pltpu.VMEM((4, 128), _i32),
        pltpu.VMEM((4, 128, W128), _f32),
        pltpu.VMEM((16,), _i32),
        pltpu.SemaphoreType.DMA,
        pltpu.SemaphoreType.DMA,
    ],
)
def _sc_aggregate(h, ssrc, sdst, qtab, zb, agg, accum, idxbuf, dbuf128, dstbuf, rows, cbuf, gsem, ssem):
    c = lax.axis_index("c")
    s = lax.axis_index("s")
    pltpu.sync_copy(qtab.at[0], cbuf)
    sv = cbuf[...]
    pltpu.sync_copy(qtab.at[1], cbuf)
    nv = cbuf[...]

    def chunk(qi, _):
        q = c * (NQ // NC) + qi
        sb = jnp.zeros((), _i32)
        nb = jnp.zeros((), _i32)
        for lane in range(NQ):
            sb = jnp.where(q == lane, sv[lane], sb)
            nb = jnp.where(q == lane, nv[lane], nb)
        pltpu.sync_copy(zb, accum.at[pl.ds(s * ZROWS, ZROWS)])
        plsc.subcore_barrier()
        lo = q * CS

        def stage(k, u):
            j = sb + s + 16 * k
            pltpu.sync_copy(ssrc.at[pl.ds(j * 128, 128)], idxbuf.at[u])
            pltpu.sync_copy(sdst.at[pl.ds(j * 128, 128)], dbuf128)
            for k8 in range(8):
                d = dbuf128[pl.ds(k8 * 16, 16)]
                dl = d - lo
                ok = (dl >= 0) & (dl < CS)
                dstbuf[u, pl.ds(k8 * 16, 16)] = jnp.where(ok, dl, CS)
            return pltpu.async_copy(h.at[idxbuf.at[u]], rows.at[u], gsem)

        nk = (nb - s + 15) // 16

        def blk4(g, _):
            gd = [stage(g * 4 + u, u) for u in range(4)]
            sd = []
            for u in range(4):
                gd[u].wait()
                sd.append(pltpu.async_copy(rows.at[u], accum.at[dstbuf.at[u]],
                                           ssem, add=True))
            for u in range(4):
                sd[u].wait()
            return 0

        lax.fori_loop(0, nk // 4, blk4, 0)

        def blk1(k, _):
            gd = stage(k, 0)
            gd.wait()
            pltpu.async_copy(rows.at[0], accum.at[dstbuf.at[0]], ssem, add=True).wait()
            return 0

        lax.fori_loop((nk // 4) * 4, nk, blk1, 0)
        plsc.subcore_barrier()
        pltpu.sync_copy(accum.at[pl.ds(s * WROWS, WROWS)],
                        agg.at[pl.ds(q * CS + s * WROWS, WROWS)])
        plsc.subcore_barrier()
        return 0

    lax.fori_loop(0, NQ // NC, chunk, 0)


@functools.partial(
    pl.kernel,
    out_type=jax.ShapeDtypeStruct((NP, W128), _f32),
    mesh=_mesh,
    scratch_types=[
        pltpu.VMEM_SHARED((ACC, W128), _f32),
        pltpu.VMEM((128,), _i32),
        pltpu.VMEM((128,), _i32),
        pltpu.VMEM((128, W128), _f32),
        pltpu.VMEM((16,), _i32),
        pltpu.SemaphoreType.DMA,
    ],
)
def _sc_degree(sdst, qtab, zb, ones_hbm, deg, accum, dbuf128, dstbuf, onesbuf, cbuf, ssem):
    c = lax.axis_index("c")
    s = lax.axis_index("s")
    pltpu.sync_copy(ones_hbm, onesbuf)
    pltpu.sync_copy(qtab.at[0], cbuf)
    sv = cbuf[...]
    pltpu.sync_copy(qtab.at[1], cbuf)
    nv = cbuf[...]

    def chunk(qi, _):
        q = c * (NQ // NC) + qi
        sb = jnp.zeros((), _i32)
        nb = jnp.zeros((), _i32)
        for lane in range(NQ):
            sb = jnp.where(q == lane, sv[lane], sb)
            nb = jnp.where(q == lane, nv[lane], nb)
        pltpu.sync_copy(zb, accum.at[pl.ds(s * ZROWS, ZROWS)])
        plsc.subcore_barrier()
        lo = q * CS

        def blk(k, _):
            j = sb + s + 16 * k
            pltpu.sync_copy(sdst.at[pl.ds(j * 128, 128)], dbuf128)
            for k8 in range(8):
                d = dbuf128[pl.ds(k8 * 16, 16)]
                dl = d - lo
                ok = (dl >= 0) & (dl < CS)
                dstbuf[pl.ds(k8 * 16, 16)] = jnp.where(ok, dl, CS)
            pltpu.async_copy(onesbuf, accum.at[dstbuf], ssem, add=True).wait()
            return 0

        nk = (nb - s + 15) // 16
        lax.fori_loop(0, nk, blk, 0)
        plsc.subcore_barrier()
        pltpu.sync_copy(accum.at[pl.ds(s * WROWS, WROWS)],
                        deg.at[pl.ds(q * CS + s * WROWS, WROWS)])
        plsc.subcore_barrier()
        return 0

    lax.fori_loop(0, NQ // NC, chunk, 0)


@functools.partial(
    pl.kernel,
    out_type=jax.ShapeDtypeStruct((NP, EMBP), _f32),
    mesh=_mesh,
    scratch_types=[
        pltpu.VMEM((112,), _i32),
        pltpu.VMEM((112, EMBP), _f32),
        pltpu.SemaphoreType.DMA,
    ],
)
def _sc_gather_emb(embp, eidx2, xg, ibuf, rows, gsem):
    c = lax.axis_index("c")
    s = lax.axis_index("s")
    w = s * NC + c

    def blk(j, _):
        r = w * 28 + j
        pltpu.sync_copy(eidx2.at[r], ibuf)
        pltpu.async_copy(embp.at[ibuf], rows, gsem).wait()
        pltpu.sync_copy(rows, xg.at[pl.ds(r * 112, 112)])
        return 0

    lax.fori_loop(0, 28, blk, 0)


@functools.partial(
    pl.kernel,
    out_type=jax.ShapeDtypeStruct((2048, W128), _f32),
    mesh=_mesh,
    scratch_types=[
        pltpu.VMEM((64,), _i32),
        pltpu.VMEM((64, W128), _f32),
        pltpu.SemaphoreType.DMA,
    ],
)
def _sc_gather_rows(h, pidx2, geg, ibuf, rows, gsem):
    c = lax.axis_index("c")
    s = lax.axis_index("s")
    w = s * NC + c
    pltpu.sync_copy(pidx2.at[w], ibuf)
    pltpu.async_copy(h.at[ibuf], rows, gsem).wait()
    pltpu.sync_copy(rows, geg.at[pl.ds(w * 64, 64)])


def _t0_body(xg_ref, pwt_ref, pb_ref, o_ref):
    x = jnp.dot(xg_ref[...], pwt_ref[...], preferred_element_type=_f32) + pb_ref[...]
    o_ref[...] = jnp.concatenate([x, jnp.zeros((x.shape[0], W128 - HID), _f32)], axis=1)


def _t1_body(agg_ref, h_ref, dg_ref, wt_ref, b_ref, wih_ref, bih_ref, whh_ref, bhh_ref, o_ref):
    h = h_ref[...][:, :HID]
    inc = jnp.dot(agg_ref[...], wt_ref[...], preferred_element_type=_f32)
    inc = inc + dg_ref[...][:, :HID] * b_ref[...]
    gi = jnp.dot(inc, wih_ref[...], preferred_element_type=_f32) + bih_ref[...]
    gh = jnp.dot(h, whh_ref[...], preferred_element_type=_f32) + bhh_ref[...]
    r = jax.nn.sigmoid(gi[:, :HID] + gh[:, :HID])
    z = jax.nn.sigmoid(gi[:, HID:2 * HID] + gh[:, HID:2 * HID])
    n = jnp.tanh(gi[:, 2 * HID:] + r * gh[:, 2 * HID:])
    hn = (1.0 - z) * n + z * h
    o_ref[...] = jnp.concatenate([hn, jnp.zeros((hn.shape[0], W128 - HID), _f32)], axis=1)


def _tf_body(ge_ref, w1_ref, b1_ref, w2_ref, b2_ref, lab_ref, logit_ref, loss_ref):
    ge = ge_ref[...]
    hc = jnp.maximum(jnp.dot(ge, w1_ref[...], preferred_element_type=_f32) + b1_ref[...], 0.0)
    lg = jnp.dot(hc, w2_ref[...], preferred_element_type=_f32) + b2_ref[...]
    pv = jax.nn.sigmoid(lg)
    logit_ref[...] = pv
    lab = lab_ref[...]
    p = jnp.clip(pv, 1e-7, 1.0 - 1e-7)
    terms = lab * jnp.log(p) + (1.0 - lab) * jnp.log(1.0 - p)
    loss_ref[0, 0] = -jnp.sum(terms) / 1024.0


_GRID = NP // 1024


def _row_spec(width):
    return pl.BlockSpec((1024, width), lambda i: (i, 0))


def _full_spec(shape):
    return pl.BlockSpec(shape, lambda i: tuple(0 for _ in shape))


_t0_call = pl.pallas_call(
    _t0_body,
    grid=(_GRID,),
    in_specs=[_row_spec(EMBP), _full_spec((EMBP, HID)), _full_spec((1, HID))],
    out_specs=_row_spec(W128),
    out_shape=jax.ShapeDtypeStruct((NP, W128), _f32),
)

_t1_call = pl.pallas_call(
    _t1_body,
    grid=(_GRID,),
    in_specs=[
        _row_spec(W128), _row_spec(W128), _row_spec(W128),
        _full_spec((W128, HID)), _full_spec((1, HID)),
        _full_spec((HID, 3 * HID)), _full_spec((1, 3 * HID)),
        _full_spec((HID, 3 * HID)), _full_spec((1, 3 * HID)),
    ],
    out_specs=_row_spec(W128),
    out_shape=jax.ShapeDtypeStruct((NP, W128), _f32),
)

_tf_call = pl.pallas_call(
    _tf_body,
    in_specs=[
        pl.BlockSpec(memory_space=pltpu.VMEM),
        pl.BlockSpec(memory_space=pltpu.VMEM),
        pl.BlockSpec(memory_space=pltpu.VMEM),
        pl.BlockSpec(memory_space=pltpu.VMEM),
        pl.BlockSpec(memory_space=pltpu.VMEM),
        pl.BlockSpec(memory_space=pltpu.VMEM),
    ],
    out_specs=(
        pl.BlockSpec(memory_space=pltpu.VMEM),
        pl.BlockSpec(memory_space=pltpu.SMEM),
    ),
    out_shape=(
        jax.ShapeDtypeStruct((1024, 1), _f32),
        jax.ShapeDtypeStruct((1, 1), _f32),
    ),
)


def kernel(emb_ind_0, emb_ind_1, adj_0, adj_1, prop_ind_0, prop_ind_1, labels,
           emb_table, proj_W, proj_b, msg_W, msg_b, gru_Wih, gru_Whh, gru_bih, gru_bhh,
           cla_W1, cla_b1, cla_W2, cla_b2):
    src = jnp.concatenate([adj_0[:, 0], adj_1[:, 0] + N])
    dst = jnp.concatenate([adj_0[:, 1], adj_1[:, 1] + N])
    emb_ind = jnp.concatenate([emb_ind_0, emb_ind_1])
    embp = jnp.pad(emb_table, ((0, 0), (0, EMBP - emb_table.shape[1])))
    pwt = jnp.pad(proj_W.T, ((0, EMBP - proj_W.shape[1]), (0, 0)))

    eidx2 = jnp.pad(emb_ind, (0, NP - NN)).reshape(NP // 112, 112)
    zb = jnp.zeros((ZROWS, W128), _f32)
    ones = jnp.ones((128, W128), _f32)

    sdst, ssrc = lax.sort((dst, src), num_keys=1)
    b = jnp.searchsorted(sdst, jnp.arange(NQ + 1, dtype=_i32) * CS)
    startblk = (b[:NQ] // 128).astype(_i32)
    nblk = ((b[1:] + 127) // 128).astype(_i32) - startblk
    qtab = jnp.stack([startblk, nblk])

    deg = _sc_degree(sdst, qtab, zb, ones)
    xg = _sc_gather_emb(embp, eidx2)
    h = _t0_call(xg, pwt, proj_b.reshape(1, HID))

    for l in range(2):
        wt = jnp.pad(msg_W[l].T, ((0, W128 - HID), (0, 0)))
        b = msg_b[l].reshape(1, HID)
        wih = gru_Wih[l].T
        bih = gru_bih[l].reshape(1, 3 * HID)
        whh = gru_Whh[l].T
        bhh = gru_bhh[l].reshape(1, 3 * HID)
        for _ in range(3):
            agg = _sc_aggregate(h, ssrc, sdst, qtab, zb)
            h = _t1_call(agg, h, deg, wt, b, wih, bih, whh, bhh)

    pidx2 = jnp.concatenate([prop_ind_0, prop_ind_1 + N]).reshape(NW, 64)
    geg = _sc_gather_rows(h, pidx2)
    ge = jnp.concatenate([geg[:1024, :HID], geg[1024:, :HID]], axis=1)

    logits, loss = _tf_call(
        ge, cla_W1.T, cla_b1.reshape(1, HID), cla_W2.T, cla_b2.reshape(1, 1),
        labels.astype(_f32).reshape(1024, 1),
    )
    return logits, loss.reshape(())


# speedup vs baseline: 6.5304x; 1.0958x over previous
import functools

import jax
import jax.numpy as jnp
from jax import lax
from jax.experimental import pallas as pl
from jax.experimental.pallas import tpu as pltpu
from jax.experimental.pallas import tpu_sc as plsc

N = 50000
NN = 2 * N
CS = 6272
NQ = 16
NP = CS * NQ
E = 800000
E2 = 2 * E
HID = 64
W128 = 128
EMBP = 128
NC, NS = 2, 16
NW = NC * NS
EW = E2 // NW
B1 = 2000
STG = 2176
CAP = 50176
NBCAP = CAP // 128
ACC = CS + 128
ZROWS = ACC // NS
WROWS = CS // NS

_mesh = plsc.VectorSubcoreMesh(core_axis_name="c", subcore_axis_name="s")
_f32 = jnp.float32
_i32 = jnp.int32


def _iota16():
    return lax.iota(_i32, 16)


@functools.partial(
    pl.kernel,
    out_type=jax.ShapeDtypeStruct((NP, W128), _f32),
    mesh=_mesh,
    scratch_types=[
        pltpu.VMEM_SHARED((ACC, W128), _f32),
        pltpu.VMEM((4, 128), _i32),
        pltpu.VMEM((128,), _i32),
        pltpu.VMEM((4, 128), _i32),
        pltpu.VMEM((4, 128, W128), _f32),
        pltpu.VMEM((16,), _i32),
        pltpu.SemaphoreType.DMA,
        pltpu.SemaphoreType.DMA,
    ],
)
def _sc_aggregate(h, ssrc, sdst, qtab, zb, agg, accum, idxbuf, dbuf128, dstbuf, rows, cbuf, gsem, ssem):
    c = lax.axis_index("c")
    s = lax.axis_index("s")
    pltpu.sync_copy(qtab.at[0], cbuf)
    sv = cbuf[...]
    pltpu.sync_copy(qtab.at[1], cbuf)
    nv = cbuf[...]

    def chunk(qi, _):
        q = c * (NQ // NC) + qi
        sb = jnp.zeros((), _i32)
        nb = jnp.zeros((), _i32)
        for lane in range(NQ):
            sb = jnp.where(q == lane, sv[lane], sb)
            nb = jnp.where(q == lane, nv[lane], nb)
        pltpu.sync_copy(zb, accum.at[pl.ds(s * ZROWS, ZROWS)])
        plsc.subcore_barrier()
        lo = q * CS

        def stage(k, u):
            j = sb + s + 16 * k
            pltpu.sync_copy(ssrc.at[pl.ds(j * 128, 128)], idxbuf.at[u])
            pltpu.sync_copy(sdst.at[pl.ds(j * 128, 128)], dbuf128)
            for k8 in range(8):
                d = dbuf128[pl.ds(k8 * 16, 16)]
                dl = d - lo
                ok = (dl >= 0) & (dl < CS)
                dstbuf[u, pl.ds(k8 * 16, 16)] = jnp.where(ok, dl, CS)
            return pltpu.async_copy(h.at[idxbuf.at[u]], rows.at[u], gsem)

        nk = (nb - s + 15) // 16

        def drain_one(u):
            pltpu.make_async_copy(h.at[pl.ds(0, 128)], rows.at[u], ssem).wait()

        def blk4(g, _):
            gd = []
            for u in range(4):
                @pl.when(g > 0)
                def _(u=u):
                    drain_one(u)
                gd.append(stage(g * 4 + u, u))
            for u in range(4):
                gd[u].wait()
                pltpu.async_copy(rows.at[u], accum.at[dstbuf.at[u]], ssem, add=True)
            return 0

        ng = nk // 4
        lax.fori_loop(0, ng, blk4, 0)
        for u in range(4):
            @pl.when(ng > 0)
            def _(u=u):
                drain_one(u)

        def blk1(k, _):
            gd = stage(k, 0)
            gd.wait()
            pltpu.async_copy(rows.at[0], accum.at[dstbuf.at[0]], ssem, add=True).wait()
            return 0

        lax.fori_loop(ng * 4, nk, blk1, 0)
        plsc.subcore_barrier()
        pltpu.sync_copy(accum.at[pl.ds(s * WROWS, WROWS)],
                        agg.at[pl.ds(q * CS + s * WROWS, WROWS)])
        plsc.subcore_barrier()
        return 0

    lax.fori_loop(0, NQ // NC, chunk, 0)


@functools.partial(
    pl.kernel,
    out_type=jax.ShapeDtypeStruct((NP, W128), _f32),
    mesh=_mesh,
    scratch_types=[
        pltpu.VMEM_SHARED((ACC, W128), _f32),
        pltpu.VMEM((128,), _i32),
        pltpu.VMEM((4, 128), _i32),
        pltpu.VMEM((128, W128), _f32),
        pltpu.VMEM((16,), _i32),
        pltpu.SemaphoreType.DMA,
    ],
)
def _sc_degree(sdst, qtab, zb, ones_hbm, deg, accum, dbuf128, dstbuf, onesbuf, cbuf, ssem):
    c = lax.axis_index("c")
    s = lax.axis_index("s")
    pltpu.sync_copy(ones_hbm, onesbuf)
    pltpu.sync_copy(qtab.at[0], cbuf)
    sv = cbuf[...]
    pltpu.sync_copy(qtab.at[1], cbuf)
    nv = cbuf[...]

    def chunk(qi, _):
        q = c * (NQ // NC) + qi
        sb = jnp.zeros((), _i32)
        nb = jnp.zeros((), _i32)
        for lane in range(NQ):
            sb = jnp.where(q == lane, sv[lane], sb)
            nb = jnp.where(q == lane, nv[lane], nb)
        pltpu.sync_copy(zb, accum.at[pl.ds(s * ZROWS, ZROWS)])
        plsc.subcore_barrier()
        lo = q * CS

        def fire(k, u):
            j = sb + s + 16 * k
            pltpu.sync_copy(sdst.at[pl.ds(j * 128, 128)], dbuf128)
            for k8 in range(8):
                d = dbuf128[pl.ds(k8 * 16, 16)]
                dl = d - lo
                ok = (dl >= 0) & (dl < CS)
                dstbuf[u, pl.ds(k8 * 16, 16)] = jnp.where(ok, dl, CS)
            pltpu.async_copy(onesbuf, accum.at[dstbuf.at[u]], ssem, add=True)

        nk = (nb - s + 15) // 16

        def blk4(g, _):
            for u in range(4):
                @pl.when(g > 0)
                def _(u=u):
                    pltpu.make_async_copy(deg.at[pl.ds(0, 128)], onesbuf, ssem).wait()
                fire(g * 4 + u, u)
            return 0

        ng = nk // 4
        lax.fori_loop(0, ng, blk4, 0)
        for u in range(4):
            @pl.when(ng > 0)
            def _():
                pltpu.make_async_copy(deg.at[pl.ds(0, 128)], onesbuf, ssem).wait()

        def blk1(k, _):
            fire(k, 0)
            pltpu.make_async_copy(deg.at[pl.ds(0, 128)], onesbuf, ssem).wait()
            return 0

        lax.fori_loop(ng * 4, nk, blk1, 0)
        plsc.subcore_barrier()
        pltpu.sync_copy(accum.at[pl.ds(s * WROWS, WROWS)],
                        deg.at[pl.ds(q * CS + s * WROWS, WROWS)])
        plsc.subcore_barrier()
        return 0

    lax.fori_loop(0, NQ // NC, chunk, 0)


@functools.partial(
    pl.kernel,
    out_type=jax.ShapeDtypeStruct((NP, EMBP), _f32),
    mesh=_mesh,
    scratch_types=[
        pltpu.VMEM((112,), _i32),
        pltpu.VMEM((112, EMBP), _f32),
        pltpu.SemaphoreType.DMA,
    ],
)
def _sc_gather_emb(embp, eidx2, xg, ibuf, rows, gsem):
    c = lax.axis_index("c")
    s = lax.axis_index("s")
    w = s * NC + c

    def blk(j, _):
        r = w * 28 + j
        pltpu.sync_copy(eidx2.at[r], ibuf)
        pltpu.async_copy(embp.at[ibuf], rows, gsem).wait()
        pltpu.sync_copy(rows, xg.at[pl.ds(r * 112, 112)])
        return 0

    lax.fori_loop(0, 28, blk, 0)


@functools.partial(
    pl.kernel,
    out_type=jax.ShapeDtypeStruct((2048, W128), _f32),
    mesh=_mesh,
    scratch_types=[
        pltpu.VMEM((64,), _i32),
        pltpu.VMEM((64, W128), _f32),
        pltpu.SemaphoreType.DMA,
    ],
)
def _sc_gather_rows(h, pidx2, geg, ibuf, rows, gsem):
    c = lax.axis_index("c")
    s = lax.axis_index("s")
    w = s * NC + c
    pltpu.sync_copy(pidx2.at[w], ibuf)
    pltpu.async_copy(h.at[ibuf], rows, gsem).wait()
    pltpu.sync_copy(rows, geg.at[pl.ds(w * 64, 64)])


def _t0_body(xg_ref, pwt_ref, pb_ref, o_ref):
    x = jnp.dot(xg_ref[...], pwt_ref[...], preferred_element_type=_f32) + pb_ref[...]
    o_ref[...] = jnp.concatenate([x, jnp.zeros((x.shape[0], W128 - HID), _f32)], axis=1)


def _t1_body(agg_ref, h_ref, dg_ref, wt_ref, b_ref, wih_ref, bih_ref, whh_ref, bhh_ref, o_ref):
    h = h_ref[...][:, :HID]
    inc = jnp.dot(agg_ref[...], wt_ref[...], preferred_element_type=_f32)
    inc = inc + dg_ref[...][:, :HID] * b_ref[...]
    gi = jnp.dot(inc, wih_ref[...], preferred_element_type=_f32) + bih_ref[...]
    gh = jnp.dot(h, whh_ref[...], preferred_element_type=_f32) + bhh_ref[...]
    r = jax.nn.sigmoid(gi[:, :HID] + gh[:, :HID])
    z = jax.nn.sigmoid(gi[:, HID:2 * HID] + gh[:, HID:2 * HID])
    n = jnp.tanh(gi[:, 2 * HID:] + r * gh[:, 2 * HID:])
    hn = (1.0 - z) * n + z * h
    o_ref[...] = jnp.concatenate([hn, jnp.zeros((hn.shape[0], W128 - HID), _f32)], axis=1)


def _tf_body(ge_ref, w1_ref, b1_ref, w2_ref, b2_ref, lab_ref, logit_ref, loss_ref):
    ge = ge_ref[...]
    hc = jnp.maximum(jnp.dot(ge, w1_ref[...], preferred_element_type=_f32) + b1_ref[...], 0.0)
    lg = jnp.dot(hc, w2_ref[...], preferred_element_type=_f32) + b2_ref[...]
    pv = jax.nn.sigmoid(lg)
    logit_ref[...] = pv
    lab = lab_ref[...]
    p = jnp.clip(pv, 1e-7, 1.0 - 1e-7)
    terms = lab * jnp.log(p) + (1.0 - lab) * jnp.log(1.0 - p)
    loss_ref[0, 0] = -jnp.sum(terms) / 1024.0


_GRID = NP // 1024


def _row_spec(width):
    return pl.BlockSpec((1024, width), lambda i: (i, 0))


def _full_spec(shape):
    return pl.BlockSpec(shape, lambda i: tuple(0 for _ in shape))


_t0_call = pl.pallas_call(
    _t0_body,
    grid=(_GRID,),
    in_specs=[_row_spec(EMBP), _full_spec((EMBP, HID)), _full_spec((1, HID))],
    out_specs=_row_spec(W128),
    out_shape=jax.ShapeDtypeStruct((NP, W128), _f32),
)

_t1_call = pl.pallas_call(
    _t1_body,
    grid=(_GRID,),
    in_specs=[
        _row_spec(W128), _row_spec(W128), _row_spec(W128),
        _full_spec((W128, HID)), _full_spec((1, HID)),
        _full_spec((HID, 3 * HID)), _full_spec((1, 3 * HID)),
        _full_spec((HID, 3 * HID)), _full_spec((1, 3 * HID)),
    ],
    out_specs=_row_spec(W128),
    out_shape=jax.ShapeDtypeStruct((NP, W128), _f32),
)

_tf_call = pl.pallas_call(
    _tf_body,
    in_specs=[
        pl.BlockSpec(memory_space=pltpu.VMEM),
        pl.BlockSpec(memory_space=pltpu.VMEM),
        pl.BlockSpec(memory_space=pltpu.VMEM),
        pl.BlockSpec(memory_space=pltpu.VMEM),
        pl.BlockSpec(memory_space=pltpu.VMEM),
        pl.BlockSpec(memory_space=pltpu.VMEM),
    ],
    out_specs=(
        pl.BlockSpec(memory_space=pltpu.VMEM),
        pl.BlockSpec(memory_space=pltpu.SMEM),
    ),
    out_shape=(
        jax.ShapeDtypeStruct((1024, 1), _f32),
        jax.ShapeDtypeStruct((1, 1), _f32),
    ),
)


def kernel(emb_ind_0, emb_ind_1, adj_0, adj_1, prop_ind_0, prop_ind_1, labels,
           emb_table, proj_W, proj_b, msg_W, msg_b, gru_Wih, gru_Whh, gru_bih, gru_bhh,
           cla_W1, cla_b1, cla_W2, cla_b2):
    src = jnp.concatenate([adj_0[:, 0], adj_1[:, 0] + N])
    dst = jnp.concatenate([adj_0[:, 1], adj_1[:, 1] + N])
    emb_ind = jnp.concatenate([emb_ind_0, emb_ind_1])
    embp = jnp.pad(emb_table, ((0, 0), (0, EMBP - emb_table.shape[1])))
    pwt = jnp.pad(proj_W.T, ((0, EMBP - proj_W.shape[1]), (0, 0)))

    eidx2 = jnp.pad(emb_ind, (0, NP - NN)).reshape(NP // 112, 112)
    zb = jnp.zeros((ZROWS, W128), _f32)
    ones = jnp.ones((128, W128), _f32)

    sdst, ssrc = lax.sort((dst, src), num_keys=1)
    b = jnp.searchsorted(sdst, jnp.arange(NQ + 1, dtype=_i32) * CS)
    startblk = (b[:NQ] // 128).astype(_i32)
    nblk = ((b[1:] + 127) // 128).astype(_i32) - startblk
    qtab = jnp.stack([startblk, nblk])

    deg = _sc_degree(sdst, qtab, zb, ones)
    xg = _sc_gather_emb(embp, eidx2)
    h = _t0_call(xg, pwt, proj_b.reshape(1, HID))

    for l in range(2):
        wt = jnp.pad(msg_W[l].T, ((0, W128 - HID), (0, 0)))
        b = msg_b[l].reshape(1, HID)
        wih = gru_Wih[l].T
        bih = gru_bih[l].reshape(1, 3 * HID)
        whh = gru_Whh[l].T
        bhh = gru_bhh[l].reshape(1, 3 * HID)
        for _ in range(3):
            agg = _sc_aggregate(h, ssrc, sdst, qtab, zb)
            h = _t1_call(agg, h, deg, wt, b, wih, bih, whh, bhh)

    pidx2 = jnp.concatenate([prop_ind_0, prop_ind_1 + N]).reshape(NW, 64)
    geg = _sc_gather_rows(h, pidx2)
    ge = jnp.concatenate([geg[:1024, :HID], geg[1024:, :HID]], axis=1)

    logits, loss = _tf_call(
        ge, cla_W1.T, cla_b1.reshape(1, HID), cla_W2.T, cla_b2.reshape(1, 1),
        labels.astype(_f32).reshape(1024, 1),
    )
    return logits, loss.reshape(())
